# Initial kernel scaffold; baseline (speedup 1.0000x reference)
import jax
import jax.numpy as jnp
from jax.experimental import pallas as pl


def kernel(x):
    raise NotImplementedError("write your pallas kernel here")


# speedup vs baseline: 15.4105x; 15.4105x over previous
import functools

import jax
import jax.numpy as jnp
from jax.experimental import pallas as pl
from jax.experimental.pallas import tpu as pltpu

_PERCENTILE = 90
_EPS = 1e-6


def _ash_body(x_ref, o_ref, *, k):
    sign = jnp.int32(-2147483648)
    xb = x_ref[...]
    i = jax.lax.bitcast_convert_type(xb, jnp.int32)
    flip = jax.lax.shift_right_arithmetic(i, 31)
    keys = i ^ (flip & jnp.int32(0x7FFFFFFF))
    rows = xb.shape[0]

    def step(b, prefix):
        bit = jnp.int32(1) << (jnp.int32(31) - b)
        cand = prefix | bit
        cand_cmp = cand ^ sign
        cnt = jnp.sum((keys >= cand_cmp).astype(jnp.int32), axis=1,
                      keepdims=True)
        return jnp.where(cnt >= k, cand, prefix)

    prefix = jax.lax.fori_loop(
        0, 32, step, jnp.zeros((rows, 1), jnp.int32))
    thr = prefix ^ sign
    mask = (keys >= thr).astype(jnp.float32)
    xp = xb * mask
    s_orig = jnp.sum(xb, axis=1, keepdims=True)
    s_prun = jnp.sum(xp, axis=1, keepdims=True)
    o_ref[...] = xp * (s_orig / (s_prun + _EPS))


@jax.jit
def kernel(x):
    B, C, H, W = x.shape
    n = H * W
    k = int(n * (1.0 - _PERCENTILE / 100.0))
    rows_total = B * C
    x2 = x.reshape(rows_total, n)
    R = 8
    out = pl.pallas_call(
        functools.partial(_ash_body, k=k),
        grid=(rows_total // R,),
        in_specs=[pl.BlockSpec((R, n), lambda i: (i, 0))],
        out_specs=pl.BlockSpec((R, n), lambda i: (i, 0)),
        out_shape=jax.ShapeDtypeStruct((rows_total, n), jnp.float32),
        compiler_params=pltpu.CompilerParams(
            dimension_semantics=("parallel",)),
    )(x2)
    return out.reshape(B, C, H, W)


# speedup vs baseline: 27.3013x; 1.7716x over previous
import functools

import jax
import jax.numpy as jnp
from jax.experimental import pallas as pl
from jax.experimental.pallas import tpu as pltpu

_PERCENTILE = 90
_EPS = 1e-6


def _ash_body(x_ref, o_ref, *, k):
    sign = jnp.int32(-2147483648)
    xb = x_ref[...]
    i = jax.lax.bitcast_convert_type(xb, jnp.int32)
    flip = jax.lax.shift_right_arithmetic(i, 31)
    keys = i ^ (flip & jnp.int32(0x7FFFFFFF))
    rows = xb.shape[0]

    n = xb.shape[1]
    nq = n // 4

    def step(b, prefix):
        bit = jnp.int32(1) << (jnp.int32(31) - b)
        cand = prefix | bit
        cand_cmp = cand ^ sign
        m = (keys >= cand_cmp).astype(jnp.int32)
        cnt = ((jnp.sum(m[:, :nq], axis=1, keepdims=True)
                + jnp.sum(m[:, nq:2 * nq], axis=1, keepdims=True))
               + (jnp.sum(m[:, 2 * nq:3 * nq], axis=1, keepdims=True)
                  + jnp.sum(m[:, 3 * nq:], axis=1, keepdims=True)))
        return jnp.where(cnt >= k, cand, prefix)

    prefix = jax.lax.fori_loop(
        0, 32, step, jnp.zeros((rows, 1), jnp.int32), unroll=2)
    thr = prefix ^ sign
    mask = (keys >= thr).astype(jnp.float32)
    xp = xb * mask
    s_orig = jnp.sum(xb, axis=1, keepdims=True)
    s_prun = jnp.sum(xp, axis=1, keepdims=True)
    o_ref[...] = xp * (s_orig / (s_prun + _EPS))


@jax.jit
def kernel(x):
    B, C, H, W = x.shape
    n = H * W
    k = int(n * (1.0 - _PERCENTILE / 100.0))
    rows_total = B * C
    x2 = x.reshape(rows_total, n)
    R = 16
    out = pl.pallas_call(
        functools.partial(_ash_body, k=k),
        grid=(rows_total // R,),
        in_specs=[pl.BlockSpec((R, n), lambda i: (i, 0))],
        out_specs=pl.BlockSpec((R, n), lambda i: (i, 0)),
        out_shape=jax.ShapeDtypeStruct((rows_total, n), jnp.float32),
        compiler_params=pltpu.CompilerParams(
            dimension_semantics=("parallel",)),
    )(x2)
    return out.reshape(B, C, H, W)


# speedup vs baseline: 30.0615x; 1.1011x over previous
import functools

import jax
import jax.numpy as jnp
from jax.experimental import pallas as pl
from jax.experimental.pallas import tpu as pltpu

_PERCENTILE = 90
_EPS = 1e-6


def _ash_body(x_ref, o_ref, *, k):
    sign = jnp.int32(-2147483648)
    xb = x_ref[...]
    i = jax.lax.bitcast_convert_type(xb, jnp.int32)
    flip = jax.lax.shift_right_arithmetic(i, 31)
    keys = i ^ (flip & jnp.int32(0x7FFFFFFF))
    rows = xb.shape[0]

    n = xb.shape[1]
    nq = n // 8

    def step(b, prefix):
        bit = jnp.int32(1) << (jnp.int32(31) - b)
        cand = prefix | bit
        cand_cmp = cand ^ sign
        m = (keys >= cand_cmp).astype(jnp.int32)
        parts = [jnp.sum(m[:, j * nq:(j + 1) * nq], axis=1, keepdims=True)
                 for j in range(8)]
        while len(parts) > 1:
            parts = [parts[i] + parts[i + 1] for i in range(0, len(parts), 2)]
        cnt = parts[0]
        return jnp.where(cnt >= k, cand, prefix)

    prefix = jax.lax.fori_loop(
        0, 32, step, jnp.zeros((rows, 1), jnp.int32), unroll=4)
    thr = prefix ^ sign
    mask = (keys >= thr).astype(jnp.float32)
    xp = xb * mask
    s_orig = jnp.sum(xb, axis=1, keepdims=True)
    s_prun = jnp.sum(xp, axis=1, keepdims=True)
    o_ref[...] = xp * (s_orig / (s_prun + _EPS))


@jax.jit
def kernel(x):
    B, C, H, W = x.shape
    n = H * W
    k = int(n * (1.0 - _PERCENTILE / 100.0))
    rows_total = B * C
    x2 = x.reshape(rows_total, n)
    R = 32
    out = pl.pallas_call(
        functools.partial(_ash_body, k=k),
        grid=(rows_total // R,),
        in_specs=[pl.BlockSpec((R, n), lambda i: (i, 0))],
        out_specs=pl.BlockSpec((R, n), lambda i: (i, 0)),
        out_shape=jax.ShapeDtypeStruct((rows_total, n), jnp.float32),
        compiler_params=pltpu.CompilerParams(
            dimension_semantics=("parallel",)),
    )(x2)
    return out.reshape(B, C, H, W)
